# baseline (speedup 1.0000x reference)
import functools

import jax
import jax.numpy as jnp
from jax.experimental import pallas as pl
from jax.experimental.pallas import tpu as pltpu

E = 16
K = 2
D = 1024
F = 768
T = 2048
BT = 256


def _moe_dense_kernel(x_ref, g_ref, w13_ref, w2_ref, o_ref):
    e = pl.program_id(1)
    x = x_ref[...]
    gw = g_ref[...]
    logits = jax.lax.dot_general(
        x.astype(jnp.bfloat16), gw.astype(jnp.bfloat16),
        (((1,), (1,)), ((), ())),
        preferred_element_type=jnp.float32)
    iota_e = jax.lax.broadcasted_iota(jnp.int32, (BT, E), 1)
    m1 = jnp.max(logits, axis=1, keepdims=True)
    i1 = jnp.min(jnp.where(logits == m1, iota_e, E), axis=1, keepdims=True)
    masked = jnp.where(iota_e == i1, -jnp.inf, logits)
    m2 = jnp.max(masked, axis=1, keepdims=True)
    i2 = jnp.min(jnp.where(masked == m2, iota_e, E), axis=1, keepdims=True)
    t = jnp.exp(m2 - m1)
    w1 = 1.0 / (1.0 + t)
    w2v = t / (1.0 + t)
    we = jnp.where(i1 == e, w1, 0.0) + jnp.where(i2 == e, w2v, 0.0)

    w13 = w13_ref[0]
    w2m = w2_ref[0]
    gu = jax.lax.dot_general(
        x, w13, (((1,), (1,)), ((), ())),
        preferred_element_type=jnp.float32)
    g = gu[:, :F]
    u = gu[:, F:]
    h = (g / (1.0 + jnp.exp(-g))) * u
    o = jax.lax.dot_general(
        h, w2m, (((1,), (1,)), ((), ())),
        preferred_element_type=jnp.float32)

    @pl.when(e == 0)
    def _():
        o_ref[...] = jnp.zeros_like(o_ref)

    o_ref[...] += o * we


@jax.jit
def kernel(hidden_states, gate_weight, w13_weight, w2_weight):
    out = pl.pallas_call(
        _moe_dense_kernel,
        grid=(T // BT, E),
        in_specs=[
            pl.BlockSpec((BT, D), lambda i, e: (i, 0)),
            pl.BlockSpec((E, D), lambda i, e: (0, 0)),
            pl.BlockSpec((1, 2 * F, D), lambda i, e: (e, 0, 0)),
            pl.BlockSpec((1, D, F), lambda i, e: (e, 0, 0)),
        ],
        out_specs=pl.BlockSpec((BT, D), lambda i, e: (i, 0)),
        out_shape=jax.ShapeDtypeStruct((T, D), jnp.float32),
    )(hidden_states, gate_weight, w13_weight, w2_weight)
    return out


# speedup vs baseline: 1.8312x; 1.8312x over previous
import functools

import jax
import jax.numpy as jnp
from jax import lax
from jax.experimental import pallas as pl
from jax.experimental.pallas import tpu as pltpu
from jax.experimental.pallas import tpu_sc as plsc

E = 16
K = 2
D = 1024
F = 768
T = 2048
BT = 256
NB = T // BT
TM = 256
NA = T * K
NROWS = NA + E * TM
NTILES = NROWS // TM
NC = 2
NS = 16
NW = NC * NS
L = 16
RPW = NROWS // NW
TPW = T // NW

_i32 = jnp.int32
_f32 = jnp.float32


def _router_kernel(x_ref, g_ref, idx_ref, w_ref, cnt_ref):
    x = x_ref[...]
    gw = g_ref[...]
    logits = jax.lax.dot_general(
        x.astype(jnp.bfloat16), gw.astype(jnp.bfloat16),
        (((1,), (1,)), ((), ())),
        preferred_element_type=_f32)
    iota_e = jax.lax.broadcasted_iota(_i32, (BT, E), 1)
    m1 = jnp.max(logits, axis=1, keepdims=True)
    i1 = jnp.min(jnp.where(logits == m1, iota_e, E), axis=1, keepdims=True)
    masked = jnp.where(iota_e == i1, -jnp.inf, logits)
    m2 = jnp.max(masked, axis=1, keepdims=True)
    i2 = jnp.min(jnp.where(masked == m2, iota_e, E), axis=1, keepdims=True)
    t = jnp.exp(m2 - m1)
    w1 = 1.0 / (1.0 + t)
    w2v = t / (1.0 + t)
    idx_ref[...] = jnp.concatenate([i1, i2], axis=1)
    w_ref[...] = jnp.concatenate([w1, w2v], axis=1)
    onehot = (iota_e == i1).astype(_i32) + (iota_e == i2).astype(_i32)
    cnt_ref[...] = jnp.sum(onehot, axis=0).reshape(1, 1, E)


def _router(hidden, gate):
    return pl.pallas_call(
        _router_kernel,
        grid=(NB,),
        in_specs=[
            pl.BlockSpec((BT, D), lambda i: (i, 0)),
            pl.BlockSpec((E, D), lambda i: (0, 0)),
        ],
        out_specs=[
            pl.BlockSpec((BT, K), lambda i: (i, 0)),
            pl.BlockSpec((BT, K), lambda i: (i, 0)),
            pl.BlockSpec((1, 1, E), lambda i: (i, 0, 0)),
        ],
        out_shape=[
            jax.ShapeDtypeStruct((T, K), _i32),
            jax.ShapeDtypeStruct((T, K), _f32),
            jax.ShapeDtypeStruct((NB, 1, E), _i32),
        ],
    )(hidden, gate)


def _sc_mesh():
    return plsc.VectorSubcoreMesh(core_axis_name="c", subcore_axis_name="s",
                                  num_cores=NC, num_subcores=NS)


def _wid():
    return lax.axis_index("c") * NS + lax.axis_index("s")


def _lane():
    return lax.broadcasted_iota(_i32, (L,), 0)


def _offsets(cnt_v):
    totals = cnt_v[pl.ds(0, L)]
    for b in range(1, NB):
        totals = totals + cnt_v[pl.ds(b * L, L)]
    padded = ((totals + (TM - 1)) >> 8) << 8
    inc = plsc.cumsum(padded)
    return totals, padded, inc


def _dispatch_body(cnt_hbm, ef_hbm, wf_hbm,
                   rowflat_hbm, roww_hbm, seg_hbm,
                   cnt_v, ef_v, wf_v, flatb, wb, negb, zerob, segb):
    wid = _wid()
    lane = _lane()
    pltpu.sync_copy(cnt_hbm, cnt_v)
    totals, padded, inc = _offsets(cnt_v)
    exc = inc - padded

    @pl.when(wid < E)
    def _():
        e = wid
        base = jnp.sum(jnp.where(lane == e, exc, 0))
        padded_e = jnp.sum(jnp.where(lane == e, padded, 0))
        pltpu.sync_copy(ef_hbm, ef_v)
        pltpu.sync_copy(wf_hbm, wf_v)
        neg1 = jnp.full((L,), -1, _i32)
        zerof = jnp.zeros((L,), _f32)

        def fill(i, c):
            flatb[pl.ds(i * L, L)] = neg1
            wb[pl.ds(i * L, L)] = zerof
            return c
        lax.fori_loop(0, T // L, fill, 0)

        def scan(i, running):
            v = ef_v[pl.ds(i * L, L)]
            m = v == e
            pref = plsc.cumsum(m.astype(_i32))
            dst = running + pref - 1
            plsc.store_scatter(flatb, [dst], i * L + lane, mask=m)
            plsc.store_scatter(wb, [dst], wf_v[pl.ds(i * L, L)], mask=m)
            return running + plsc.all_reduce_population_count(m)
        lax.fori_loop(0, NA // L, scan, jnp.zeros((L,), _i32))

        def dma(c, s):
            off = pl.multiple_of(base + c * TM, TM)
            pltpu.sync_copy(flatb.at[pl.ds(c * TM, TM)],
                            rowflat_hbm.at[pl.ds(off, TM)])
            pltpu.sync_copy(wb.at[pl.ds(c * TM, TM)],
                            roww_hbm.at[pl.ds(off, TM)])
            return s
        lax.fori_loop(0, padded_e >> 8, dma, 0)

    @pl.when(wid == E)
    def _():
        segb[pl.ds(0, L)] = exc >> 8
        segb[pl.ds(L, L)] = padded >> 8
        pltpu.sync_copy(segb, seg_hbm)

    @pl.when(wid == E + 1)
    def _():
        used_tiles = jnp.sum(jnp.where(lane == E - 1, inc, 0)) >> 8
        neg1 = jnp.full((L,), -1, _i32)
        zerof = jnp.zeros((L,), _f32)
        for i in range(TM // L):
            negb[pl.ds(i * L, L)] = neg1
            zerob[pl.ds(i * L, L)] = zerof

        def tail(c, s):
            off = pl.multiple_of(c * TM, TM)
            pltpu.sync_copy(negb, rowflat_hbm.at[pl.ds(off, TM)])
            pltpu.sync_copy(zerob, roww_hbm.at[pl.ds(off, TM)])
            return s
        lax.fori_loop(used_tiles, NTILES, tail, 0)


def _dispatch(cnt128, eflat, wflat):
    return pl.kernel(
        _dispatch_body,
        out_type=(
            jax.ShapeDtypeStruct((NROWS,), _i32),
            jax.ShapeDtypeStruct((NROWS,), _f32),
            jax.ShapeDtypeStruct((2 * E,), _i32),
        ),
        mesh=_sc_mesh(),
        compiler_params=pltpu.CompilerParams(needs_layout_passes=False),
        scratch_types=[
            pltpu.VMEM((NB * E,), _i32),
            pltpu.VMEM((NA,), _i32),
            pltpu.VMEM((NA,), _f32),
            pltpu.VMEM((T,), _i32),
            pltpu.VMEM((T,), _f32),
            pltpu.VMEM((TM,), _i32),
            pltpu.VMEM((TM,), _f32),
            pltpu.VMEM((2 * E,), _i32),
        ],
    )(cnt128, eflat, wflat)


def _gather_body(hid_hbm, rowflat_hbm, xs_hbm, pos_hbm,
                 rf_v, tok_v, rows_v, rfall, posb, sem):
    wid = _wid()
    lane = _lane()
    base_r = pl.multiple_of(wid * RPW, 64)
    pltpu.sync_copy(rowflat_hbm.at[pl.ds(base_r, RPW)], rf_v)
    for j in range(RPW // L):
        v = rf_v[pl.ds(j * L, L)]
        pvec = base_r + j * L + lane
        tok_v[pl.ds(j * L, L)] = jnp.where(v < 0, pvec & (T - 1), v >> 1)
    for c in range(RPW // 64):
        anyreal = jnp.zeros((L,), _i32)
        for j in range(4):
            anyreal = jnp.maximum(anyreal, rf_v[pl.ds(c * 64 + j * L, L)])
        has_real = jnp.max(anyreal) >= 0

        @pl.when(has_real)
        def _():
            cp = pltpu.async_copy(
                hid_hbm.at[tok_v.at[pl.ds(c * 64, 64)]], rows_v, sem)
            cp.wait()
            pltpu.sync_copy(
                rows_v,
                xs_hbm.at[pl.ds(pl.multiple_of(base_r + c * 64, 64), 64)])

    @pl.when(wid == 0)
    def _():
        pltpu.sync_copy(rowflat_hbm, rfall)

        def inv(i, s):
            v = rfall[pl.ds(i * L, L)]
            plsc.store_scatter(posb, [v], i * L + lane, mask=v >= 0)
            return s
        lax.fori_loop(0, NROWS // L, inv, 0)
        pltpu.sync_copy(posb, pos_hbm)


def _gather_rows(hidden, rowflat):
    return pl.kernel(
        _gather_body,
        out_type=(
            jax.ShapeDtypeStruct((NROWS, D), _f32),
            jax.ShapeDtypeStruct((NA,), _i32),
        ),
        mesh=_sc_mesh(),
        compiler_params=pltpu.CompilerParams(needs_layout_passes=False),
        scratch_types=[
            pltpu.VMEM((RPW,), _i32),
            pltpu.VMEM((RPW,), _i32),
            pltpu.VMEM((64, D), _f32),
            pltpu.VMEM((NROWS,), _i32),
            pltpu.VMEM((NA,), _i32),
            pltpu.SemaphoreType.DMA,
        ],
    )(hidden, rowflat)


def _ffn_kernel(seg_ref, x_hbm, w13_ref, w2_ref, rw_hbm, o_hbm,
                xbuf, obuf, rwbuf, xsem, osem, rwsem):
    e = pl.program_id(0)
    t0 = seg_ref[e]
    nt = seg_ref[E + e]
    w13 = w13_ref[0].astype(jnp.bfloat16)
    w2m = w2_ref[0].astype(jnp.bfloat16)

    def x_copy(c, slot):
        row = pl.multiple_of((t0 + c) * TM, TM)
        return pltpu.make_async_copy(
            x_hbm.at[pl.ds(row, TM)], xbuf.at[slot], xsem.at[slot])

    def rw_copy(c, slot):
        row = pl.multiple_of((t0 + c) * TM, TM)
        return pltpu.make_async_copy(
            rw_hbm.at[pl.ds(row, TM)], rwbuf.at[slot], rwsem.at[slot])

    def o_copy(c, slot):
        row = pl.multiple_of((t0 + c) * TM, TM)
        return pltpu.make_async_copy(
            obuf.at[slot], o_hbm.at[pl.ds(row, TM)], osem.at[slot])

    @pl.when(nt > 0)
    def _():
        x_copy(0, 0).start()
        rw_copy(0, 0).start()

        def step(c, s):
            slot = lax.rem(c, 2)

            @pl.when(c + 1 < nt)
            def _():
                x_copy(c + 1, 1 - slot).start()
                rw_copy(c + 1, 1 - slot).start()

            x_copy(c, slot).wait()
            rw_copy(c, slot).wait()
            x = xbuf[slot].astype(jnp.bfloat16)
            gu = jax.lax.dot_general(
                x, w13, (((1,), (1,)), ((), ())),
                preferred_element_type=_f32)
            g = gu[:, :F]
            u = gu[:, F:]
            h = ((g / (1.0 + jnp.exp(-g))) * u).astype(jnp.bfloat16)
            o = jax.lax.dot_general(
                h, w2m, (((1,), (1,)), ((), ())),
                preferred_element_type=_f32)

            @pl.when(c >= 2)
            def _():
                o_copy(c - 2, slot).wait()

            obuf[slot] = o * rwbuf[slot]
            o_copy(c, slot).start()
            return s

        lax.fori_loop(0, nt, step, 0)

        @pl.when(nt >= 2)
        def _():
            o_copy(nt - 2, lax.rem(nt, 2)).wait()
        o_copy(nt - 1, lax.rem(nt + 1, 2)).wait()


def _ffn(seg_info, xsorted, w13, w2, roww):
    grid_spec = pltpu.PrefetchScalarGridSpec(
        num_scalar_prefetch=1,
        grid=(E,),
        in_specs=[
            pl.BlockSpec(memory_space=pl.ANY),
            pl.BlockSpec((1, 2 * F, D), lambda e, seg: (e, 0, 0)),
            pl.BlockSpec((1, D, F), lambda e, seg: (e, 0, 0)),
            pl.BlockSpec(memory_space=pl.ANY),
        ],
        out_specs=pl.BlockSpec(memory_space=pl.ANY),
        scratch_shapes=[
            pltpu.VMEM((2, TM, D), _f32),
            pltpu.VMEM((2, TM, D), _f32),
            pltpu.VMEM((2, TM, 1), _f32),
            pltpu.SemaphoreType.DMA((2,)),
            pltpu.SemaphoreType.DMA((2,)),
            pltpu.SemaphoreType.DMA((2,)),
        ],
    )
    return pl.pallas_call(
        _ffn_kernel,
        grid_spec=grid_spec,
        out_shape=jax.ShapeDtypeStruct((NROWS, D), _f32),
    )(seg_info, xsorted, w13, w2, roww)


def _combine_body(y_hbm, pos_hbm, out_hbm, posv, rows_v, obuf, sem):
    wid = _wid()
    pltpu.sync_copy(
        pos_hbm.at[pl.ds(pl.multiple_of(wid * K * TPW, K * TPW), K * TPW)],
        posv)
    for c in range(K * TPW // 64):
        cp = pltpu.async_copy(
            y_hbm.at[posv.at[pl.ds(c * 64, 64)]], rows_v, sem)
        cp.wait()

        @plsc.parallel_loop(0, 32 * (D // L), unroll=8)
        def addloop(i):
            p = i >> 6
            dd = (i & 63) * L
            a = rows_v[p * 2, pl.ds(dd, L)]
            b = rows_v[p * 2 + 1, pl.ds(dd, L)]
            obuf[p, pl.ds(dd, L)] = a + b

        pltpu.sync_copy(
            obuf, out_hbm.at[pl.ds(pl.multiple_of(wid * TPW + c * 32, 32), 32)])


def _combine(y, pos):
    return pl.kernel(
        _combine_body,
        out_type=jax.ShapeDtypeStruct((T, D), _f32),
        mesh=_sc_mesh(),
        compiler_params=pltpu.CompilerParams(needs_layout_passes=False),
        scratch_types=[
            pltpu.VMEM((K * TPW,), _i32),
            pltpu.VMEM((64, D), _f32),
            pltpu.VMEM((32, D), _f32),
            pltpu.SemaphoreType.DMA,
        ],
    )(y, pos)


@jax.jit
def kernel(hidden_states, gate_weight, w13_weight, w2_weight):
    idx, w, cnt = _router(hidden_states, gate_weight)
    cnt128 = cnt.reshape(NB * E)
    eflat = idx.reshape(NA)
    wflat = w.reshape(NA)
    rowflat, roww, seg_info = _dispatch(cnt128, eflat, wflat)
    xsorted, pos = _gather_rows(hidden_states, rowflat)
    y = _ffn(seg_info, xsorted, w13_weight, w2_weight,
             roww.reshape(NROWS, 1))
    return _combine(y, pos)


# speedup vs baseline: 2.0971x; 1.1452x over previous
import functools

import jax
import jax.numpy as jnp
from jax import lax
from jax.experimental import pallas as pl
from jax.experimental.pallas import tpu as pltpu
from jax.experimental.pallas import tpu_sc as plsc

E = 16
K = 2
D = 1024
F = 768
T = 2048
BT = 256
NB = T // BT
TM = 256
NA = T * K
NROWS = NA + E * TM
NTILES = NROWS // TM
NC = 2
NS = 16
NW = NC * NS
L = 16
RPW = NROWS // NW
TPW = T // NW

_i32 = jnp.int32
_f32 = jnp.float32


def _router_kernel(x_ref, g_ref, idx_ref, w_ref, cnt_ref):
    x = x_ref[...]
    gw = g_ref[...]
    logits = jax.lax.dot_general(
        x.astype(jnp.bfloat16), gw.astype(jnp.bfloat16),
        (((1,), (1,)), ((), ())),
        preferred_element_type=_f32)
    iota_e = jax.lax.broadcasted_iota(_i32, (BT, E), 1)
    m1 = jnp.max(logits, axis=1, keepdims=True)
    i1 = jnp.min(jnp.where(logits == m1, iota_e, E), axis=1, keepdims=True)
    masked = jnp.where(iota_e == i1, -jnp.inf, logits)
    m2 = jnp.max(masked, axis=1, keepdims=True)
    i2 = jnp.min(jnp.where(masked == m2, iota_e, E), axis=1, keepdims=True)
    t = jnp.exp(m2 - m1)
    w1 = 1.0 / (1.0 + t)
    w2v = t / (1.0 + t)
    idx_ref[...] = jnp.concatenate([i1, i2], axis=1)
    w_ref[...] = jnp.concatenate([w1, w2v], axis=1)
    onehot = (iota_e == i1).astype(_i32) + (iota_e == i2).astype(_i32)
    cnt_ref[...] = jnp.sum(onehot, axis=0).reshape(1, 1, E)


def _router(hidden, gate):
    return pl.pallas_call(
        _router_kernel,
        grid=(NB,),
        in_specs=[
            pl.BlockSpec((BT, D), lambda i: (i, 0)),
            pl.BlockSpec((E, D), lambda i: (0, 0)),
        ],
        out_specs=[
            pl.BlockSpec((BT, K), lambda i: (i, 0)),
            pl.BlockSpec((BT, K), lambda i: (i, 0)),
            pl.BlockSpec((1, 1, E), lambda i: (i, 0, 0)),
        ],
        out_shape=[
            jax.ShapeDtypeStruct((T, K), _i32),
            jax.ShapeDtypeStruct((T, K), _f32),
            jax.ShapeDtypeStruct((NB, 1, E), _i32),
        ],
    )(hidden, gate)


def _sc_mesh():
    return plsc.VectorSubcoreMesh(core_axis_name="c", subcore_axis_name="s",
                                  num_cores=NC, num_subcores=NS)


def _wid():
    return lax.axis_index("c") * NS + lax.axis_index("s")


def _lane():
    return lax.broadcasted_iota(_i32, (L,), 0)


def _offsets(cnt_v):
    totals = cnt_v[pl.ds(0, L)]
    for b in range(1, NB):
        totals = totals + cnt_v[pl.ds(b * L, L)]
    padded = ((totals + (TM - 1)) >> 8) << 8
    inc = plsc.cumsum(padded)
    return totals, padded, inc


def _dispatch_body(cnt_hbm, ef_hbm, wf_hbm,
                   rowflat_hbm, roww_hbm, seg_hbm,
                   cnt_v, ef_v, wf_v, flatb, wb, negb, zerob, segb):
    wid = _wid()
    lane = _lane()
    pltpu.sync_copy(cnt_hbm, cnt_v)
    totals, padded, inc = _offsets(cnt_v)
    exc = inc - padded

    @pl.when(wid < E)
    def _():
        e = wid
        base = jnp.sum(jnp.where(lane == e, exc, 0))
        padded_e = jnp.sum(jnp.where(lane == e, padded, 0))
        pltpu.sync_copy(ef_hbm, ef_v)
        pltpu.sync_copy(wf_hbm, wf_v)
        neg1 = jnp.full((L,), -1, _i32)
        zerof = jnp.zeros((L,), _f32)

        def fill(i, c):
            flatb[pl.ds(i * L, L)] = neg1
            wb[pl.ds(i * L, L)] = zerof
            return c
        lax.fori_loop(0, T // L, fill, 0)

        def scan(i, running):
            v = ef_v[pl.ds(i * L, L)]
            m = v == e
            pref = plsc.cumsum(m.astype(_i32))
            dst = running + pref - 1
            plsc.store_scatter(flatb, [dst], i * L + lane, mask=m)
            plsc.store_scatter(wb, [dst], wf_v[pl.ds(i * L, L)], mask=m)
            return running + plsc.all_reduce_population_count(m)
        lax.fori_loop(0, NA // L, scan, jnp.zeros((L,), _i32))

        def dma(c, s):
            off = pl.multiple_of(base + c * TM, TM)
            pltpu.sync_copy(flatb.at[pl.ds(c * TM, TM)],
                            rowflat_hbm.at[pl.ds(off, TM)])
            pltpu.sync_copy(wb.at[pl.ds(c * TM, TM)],
                            roww_hbm.at[pl.ds(off, TM)])
            return s
        lax.fori_loop(0, padded_e >> 8, dma, 0)

    @pl.when(wid == E)
    def _():
        segb[pl.ds(0, L)] = exc >> 8
        segb[pl.ds(L, L)] = padded >> 8
        pltpu.sync_copy(segb, seg_hbm)

    @pl.when(wid == E + 1)
    def _():
        used_tiles = jnp.sum(jnp.where(lane == E - 1, inc, 0)) >> 8
        neg1 = jnp.full((L,), -1, _i32)
        zerof = jnp.zeros((L,), _f32)
        for i in range(TM // L):
            negb[pl.ds(i * L, L)] = neg1
            zerob[pl.ds(i * L, L)] = zerof

        def tail(c, s):
            off = pl.multiple_of(c * TM, TM)
            pltpu.sync_copy(negb, rowflat_hbm.at[pl.ds(off, TM)])
            pltpu.sync_copy(zerob, roww_hbm.at[pl.ds(off, TM)])
            return s
        lax.fori_loop(used_tiles, NTILES, tail, 0)


def _dispatch(cnt128, eflat, wflat):
    return pl.kernel(
        _dispatch_body,
        out_type=(
            jax.ShapeDtypeStruct((NROWS,), _i32),
            jax.ShapeDtypeStruct((NROWS,), _f32),
            jax.ShapeDtypeStruct((2 * E,), _i32),
        ),
        mesh=_sc_mesh(),
        compiler_params=pltpu.CompilerParams(needs_layout_passes=False),
        scratch_types=[
            pltpu.VMEM((NB * E,), _i32),
            pltpu.VMEM((NA,), _i32),
            pltpu.VMEM((NA,), _f32),
            pltpu.VMEM((T,), _i32),
            pltpu.VMEM((T,), _f32),
            pltpu.VMEM((TM,), _i32),
            pltpu.VMEM((TM,), _f32),
            pltpu.VMEM((2 * E,), _i32),
        ],
    )(cnt128, eflat, wflat)


def _gather_body(hid_hbm, rowflat_hbm, xs_hbm, pos_hbm,
                 rf_v, tok_v, rows_v, rfall, posb, sem):
    wid = _wid()
    lane = _lane()
    base_r = pl.multiple_of(wid * RPW, 64)
    pltpu.sync_copy(rowflat_hbm.at[pl.ds(base_r, RPW)], rf_v)
    for j in range(RPW // L):
        v = rf_v[pl.ds(j * L, L)]
        pvec = base_r + j * L + lane
        tok_v[pl.ds(j * L, L)] = jnp.where(v < 0, pvec & (T - 1), v >> 1)
    for c in range(RPW // 64):
        anyreal = jnp.zeros((L,), _i32)
        for j in range(4):
            anyreal = jnp.maximum(anyreal, rf_v[pl.ds(c * 64 + j * L, L)])
        has_real = jnp.max(anyreal) >= 0

        @pl.when(has_real)
        def _():
            cp = pltpu.async_copy(
                hid_hbm.at[tok_v.at[pl.ds(c * 64, 64)]], rows_v, sem)
            cp.wait()
            pltpu.sync_copy(
                rows_v,
                xs_hbm.at[pl.ds(pl.multiple_of(base_r + c * 64, 64), 64)])

    @pl.when(wid == 0)
    def _():
        pltpu.sync_copy(rowflat_hbm, rfall)

        def inv(i, s):
            v = rfall[pl.ds(i * L, L)]
            plsc.store_scatter(posb, [v], i * L + lane, mask=v >= 0)
            return s
        lax.fori_loop(0, NROWS // L, inv, 0)
        pltpu.sync_copy(posb, pos_hbm)


def _gather_rows(hidden, rowflat):
    return pl.kernel(
        _gather_body,
        out_type=(
            jax.ShapeDtypeStruct((NROWS, D), _f32),
            jax.ShapeDtypeStruct((NA,), _i32),
        ),
        mesh=_sc_mesh(),
        compiler_params=pltpu.CompilerParams(needs_layout_passes=False),
        scratch_types=[
            pltpu.VMEM((RPW,), _i32),
            pltpu.VMEM((RPW,), _i32),
            pltpu.VMEM((64, D), _f32),
            pltpu.VMEM((NROWS,), _i32),
            pltpu.VMEM((NA,), _i32),
            pltpu.SemaphoreType.DMA,
        ],
    )(hidden, rowflat)


def _ffn_kernel(seg_ref, x_hbm, w13_hbm, w2_hbm, rw_hbm, o_hbm,
                w13buf, w2buf, xbuf, obuf, rwbuf,
                wsem13, wsem2, xsem, osem, rwsem):
    def w_copies(e, slot):
        return (pltpu.make_async_copy(
                    w13_hbm.at[e], w13buf.at[slot], wsem13.at[slot]),
                pltpu.make_async_copy(
                    w2_hbm.at[e], w2buf.at[slot], wsem2.at[slot]))

    def x_copy(t, slot):
        row = pl.multiple_of(t * TM, TM)
        return pltpu.make_async_copy(
            x_hbm.at[pl.ds(row, TM)], xbuf.at[slot], xsem.at[slot])

    def rw_copy(t, slot):
        row = pl.multiple_of(t * TM, TM)
        return pltpu.make_async_copy(
            rw_hbm.at[pl.ds(row, TM)], rwbuf.at[slot], rwsem.at[slot])

    def o_copy(t, slot):
        row = pl.multiple_of(t * TM, TM)
        return pltpu.make_async_copy(
            obuf.at[slot], o_hbm.at[pl.ds(row, TM)], osem.at[slot])

    for cp in w_copies(0, 0):
        cp.start()

    def expert_body(e, carry):
        slot = lax.rem(e, 2)
        t0 = seg_ref[e]
        nt = seg_ref[E + e]

        @pl.when(nt > 0)
        def _():
            x_copy(t0, 0).start()
            rw_copy(t0, 0).start()

        @pl.when(e + 1 < E)
        def _():
            for cp in w_copies(e + 1, 1 - slot):
                cp.start()

        for cp in w_copies(e, slot):
            cp.wait()
        w13b = w13buf[slot].astype(jnp.bfloat16)
        w2b = w2buf[slot].astype(jnp.bfloat16)

        @pl.when(nt > 0)
        def _():
            def step(c, s):
                xslot = lax.rem(c, 2)

                @pl.when(c + 1 < nt)
                def _():
                    x_copy(t0 + c + 1, 1 - xslot).start()
                    rw_copy(t0 + c + 1, 1 - xslot).start()

                x_copy(t0 + c, xslot).wait()
                rw_copy(t0 + c, xslot).wait()
                x = xbuf[xslot].astype(jnp.bfloat16)
                gu = jax.lax.dot_general(
                    x, w13b, (((1,), (1,)), ((), ())),
                    preferred_element_type=_f32)
                g = gu[:, :F]
                u = gu[:, F:]
                h = ((g / (1.0 + jnp.exp(-g))) * u).astype(jnp.bfloat16)
                o = jax.lax.dot_general(
                    h, w2b, (((1,), (1,)), ((), ())),
                    preferred_element_type=_f32)

                @pl.when(c >= 2)
                def _():
                    o_copy(t0 + c - 2, xslot).wait()

                obuf[xslot] = o * rwbuf[xslot]
                o_copy(t0 + c, xslot).start()
                return s

            lax.fori_loop(0, nt, step, 0)

            @pl.when(nt >= 2)
            def _():
                o_copy(t0 + nt - 2, lax.rem(nt, 2)).wait()
            o_copy(t0 + nt - 1, lax.rem(nt + 1, 2)).wait()

        return carry

    lax.fori_loop(0, E, expert_body, 0)


def _ffn(seg_info, xsorted, w13, w2, roww):
    grid_spec = pltpu.PrefetchScalarGridSpec(
        num_scalar_prefetch=1,
        grid=(1,),
        in_specs=[
            pl.BlockSpec(memory_space=pl.ANY),
            pl.BlockSpec(memory_space=pl.ANY),
            pl.BlockSpec(memory_space=pl.ANY),
            pl.BlockSpec(memory_space=pl.ANY),
        ],
        out_specs=pl.BlockSpec(memory_space=pl.ANY),
        scratch_shapes=[
            pltpu.VMEM((2, 2 * F, D), _f32),
            pltpu.VMEM((2, D, F), _f32),
            pltpu.VMEM((2, TM, D), _f32),
            pltpu.VMEM((2, TM, D), _f32),
            pltpu.VMEM((2, TM, 1), _f32),
            pltpu.SemaphoreType.DMA((2,)),
            pltpu.SemaphoreType.DMA((2,)),
            pltpu.SemaphoreType.DMA((2,)),
            pltpu.SemaphoreType.DMA((2,)),
            pltpu.SemaphoreType.DMA((2,)),
        ],
    )
    return pl.pallas_call(
        _ffn_kernel,
        grid_spec=grid_spec,
        out_shape=jax.ShapeDtypeStruct((NROWS, D), _f32),
    )(seg_info, xsorted, w13, w2, roww)


def _combine_body(y_hbm, pos_hbm, out_hbm, posv, rows_v, obuf, sem):
    wid = _wid()
    pltpu.sync_copy(
        pos_hbm.at[pl.ds(pl.multiple_of(wid * K * TPW, K * TPW), K * TPW)],
        posv)
    for c in range(K * TPW // 64):
        cp = pltpu.async_copy(
            y_hbm.at[posv.at[pl.ds(c * 64, 64)]], rows_v, sem)
        cp.wait()

        @plsc.parallel_loop(0, 32 * (D // L), unroll=8)
        def addloop(i):
            p = i >> 6
            dd = (i & 63) * L
            a = rows_v[p * 2, pl.ds(dd, L)]
            b = rows_v[p * 2 + 1, pl.ds(dd, L)]
            obuf[p, pl.ds(dd, L)] = a + b

        pltpu.sync_copy(
            obuf, out_hbm.at[pl.ds(pl.multiple_of(wid * TPW + c * 32, 32), 32)])


def _combine(y, pos):
    return pl.kernel(
        _combine_body,
        out_type=jax.ShapeDtypeStruct((T, D), _f32),
        mesh=_sc_mesh(),
        compiler_params=pltpu.CompilerParams(needs_layout_passes=False),
        scratch_types=[
            pltpu.VMEM((K * TPW,), _i32),
            pltpu.VMEM((64, D), _f32),
            pltpu.VMEM((32, D), _f32),
            pltpu.SemaphoreType.DMA,
        ],
    )(y, pos)


@jax.jit
def kernel(hidden_states, gate_weight, w13_weight, w2_weight):
    idx, w, cnt = _router(hidden_states, gate_weight)
    cnt128 = cnt.reshape(NB * E)
    eflat = idx.reshape(NA)
    wflat = w.reshape(NA)
    rowflat, roww, seg_info = _dispatch(cnt128, eflat, wflat)
    xsorted, pos = _gather_rows(hidden_states, rowflat)
    y = _ffn(seg_info, xsorted, w13_weight, w2_weight,
             roww.reshape(NROWS, 1))
    return _combine(y, pos)


# speedup vs baseline: 2.2251x; 1.0610x over previous
import functools

import jax
import jax.numpy as jnp
from jax import lax
from jax.experimental import pallas as pl
from jax.experimental.pallas import tpu as pltpu
from jax.experimental.pallas import tpu_sc as plsc

E = 16
K = 2
D = 1024
F = 768
T = 2048
BT = 256
NB = T // BT
TM = 256
NA = T * K
NROWS = NA + E * TM
NTILES = NROWS // TM
NC = 2
NS = 16
NW = NC * NS
L = 16
RPW = NROWS // NW
TPW = T // NW

_i32 = jnp.int32
_f32 = jnp.float32


def _router_kernel(x_ref, g_ref, idx_ref, w_ref, cnt_ref):
    x = x_ref[...]
    gw = g_ref[...]
    logits = jax.lax.dot_general(
        x.astype(jnp.bfloat16), gw.astype(jnp.bfloat16),
        (((1,), (1,)), ((), ())),
        preferred_element_type=_f32)
    iota_e = jax.lax.broadcasted_iota(_i32, (BT, E), 1)
    m1 = jnp.max(logits, axis=1, keepdims=True)
    i1 = jnp.min(jnp.where(logits == m1, iota_e, E), axis=1, keepdims=True)
    masked = jnp.where(iota_e == i1, -jnp.inf, logits)
    m2 = jnp.max(masked, axis=1, keepdims=True)
    i2 = jnp.min(jnp.where(masked == m2, iota_e, E), axis=1, keepdims=True)
    t = jnp.exp(m2 - m1)
    w1 = 1.0 / (1.0 + t)
    w2v = t / (1.0 + t)
    idx_ref[...] = jnp.concatenate([i1, i2], axis=1)
    w_ref[...] = jnp.concatenate([w1, w2v], axis=1)
    onehot = (iota_e == i1).astype(_i32) + (iota_e == i2).astype(_i32)
    cnt_ref[...] = jnp.sum(onehot, axis=0).reshape(1, 1, E)


def _router(hidden, gate):
    return pl.pallas_call(
        _router_kernel,
        grid=(NB,),
        in_specs=[
            pl.BlockSpec((BT, D), lambda i: (i, 0)),
            pl.BlockSpec((E, D), lambda i: (0, 0)),
        ],
        out_specs=[
            pl.BlockSpec((BT, K), lambda i: (i, 0)),
            pl.BlockSpec((BT, K), lambda i: (i, 0)),
            pl.BlockSpec((1, 1, E), lambda i: (i, 0, 0)),
        ],
        out_shape=[
            jax.ShapeDtypeStruct((T, K), _i32),
            jax.ShapeDtypeStruct((T, K), _f32),
            jax.ShapeDtypeStruct((NB, 1, E), _i32),
        ],
    )(hidden, gate)


def _sc_mesh():
    return plsc.VectorSubcoreMesh(core_axis_name="c", subcore_axis_name="s",
                                  num_cores=NC, num_subcores=NS)


def _wid():
    return lax.axis_index("c") * NS + lax.axis_index("s")


def _lane():
    return lax.broadcasted_iota(_i32, (L,), 0)


def _offsets(cnt_v):
    totals = cnt_v[pl.ds(0, L)]
    for b in range(1, NB):
        totals = totals + cnt_v[pl.ds(b * L, L)]
    padded = ((totals + (TM - 1)) >> 8) << 8
    inc = plsc.cumsum(padded)
    return totals, padded, inc


def _dispatch_body(cnt_hbm, ef_hbm, wf_hbm,
                   rowflat_hbm, roww_hbm, seg_hbm,
                   cnt_v, ef_v, wf_v, flatb, wb, negb, zerob, segb):
    wid = _wid()
    lane = _lane()
    pltpu.sync_copy(cnt_hbm, cnt_v)
    totals, padded, inc = _offsets(cnt_v)
    exc = inc - padded

    @pl.when(wid < E)
    def _():
        e = wid
        base = jnp.sum(jnp.where(lane == e, exc, 0))
        padded_e = jnp.sum(jnp.where(lane == e, padded, 0))
        pltpu.sync_copy(ef_hbm, ef_v)
        pltpu.sync_copy(wf_hbm, wf_v)
        neg1 = jnp.full((L,), -1, _i32)
        zerof = jnp.zeros((L,), _f32)

        def fill(i, c):
            flatb[pl.ds(i * L, L)] = neg1
            wb[pl.ds(i * L, L)] = zerof
            return c
        lax.fori_loop(0, T // L, fill, 0)

        def scan(i, running):
            v = ef_v[pl.ds(i * L, L)]
            m = v == e
            pref = plsc.cumsum(m.astype(_i32))
            dst = running + pref - 1
            plsc.store_scatter(flatb, [dst], i * L + lane, mask=m)
            plsc.store_scatter(wb, [dst], wf_v[pl.ds(i * L, L)], mask=m)
            return running + plsc.all_reduce_population_count(m)
        lax.fori_loop(0, NA // L, scan, jnp.zeros((L,), _i32))

        def dma(c, s):
            off = pl.multiple_of(base + c * TM, TM)
            pltpu.sync_copy(flatb.at[pl.ds(c * TM, TM)],
                            rowflat_hbm.at[pl.ds(off, TM)])
            pltpu.sync_copy(wb.at[pl.ds(c * TM, TM)],
                            roww_hbm.at[pl.ds(off, TM)])
            return s
        lax.fori_loop(0, padded_e >> 8, dma, 0)

    @pl.when(wid == E)
    def _():
        segb[pl.ds(0, L)] = exc >> 8
        segb[pl.ds(L, L)] = padded >> 8
        pltpu.sync_copy(segb, seg_hbm)

    @pl.when(wid == E + 1)
    def _():
        used_tiles = jnp.sum(jnp.where(lane == E - 1, inc, 0)) >> 8
        neg1 = jnp.full((L,), -1, _i32)
        zerof = jnp.zeros((L,), _f32)
        for i in range(TM // L):
            negb[pl.ds(i * L, L)] = neg1
            zerob[pl.ds(i * L, L)] = zerof

        def tail(c, s):
            off = pl.multiple_of(c * TM, TM)
            pltpu.sync_copy(negb, rowflat_hbm.at[pl.ds(off, TM)])
            pltpu.sync_copy(zerob, roww_hbm.at[pl.ds(off, TM)])
            return s
        lax.fori_loop(used_tiles, NTILES, tail, 0)


def _dispatch(cnt128, eflat, wflat):
    return pl.kernel(
        _dispatch_body,
        out_type=(
            jax.ShapeDtypeStruct((NROWS,), _i32),
            jax.ShapeDtypeStruct((NROWS,), _f32),
            jax.ShapeDtypeStruct((2 * E,), _i32),
        ),
        mesh=_sc_mesh(),
        compiler_params=pltpu.CompilerParams(needs_layout_passes=False),
        scratch_types=[
            pltpu.VMEM((NB * E,), _i32),
            pltpu.VMEM((NA,), _i32),
            pltpu.VMEM((NA,), _f32),
            pltpu.VMEM((T,), _i32),
            pltpu.VMEM((T,), _f32),
            pltpu.VMEM((TM,), _i32),
            pltpu.VMEM((TM,), _f32),
            pltpu.VMEM((2 * E,), _i32),
        ],
    )(cnt128, eflat, wflat)


def _gather_body(hid_hbm, rowflat_hbm, xs_hbm, pos_hbm,
                 rf_v, tok_v, rows_v, rfall, posb, sem):
    wid = _wid()
    lane = _lane()
    base_r = pl.multiple_of(wid * RPW, 64)
    pltpu.sync_copy(rowflat_hbm.at[pl.ds(base_r, RPW)], rf_v)
    for j in range(RPW // L):
        v = rf_v[pl.ds(j * L, L)]
        pvec = base_r + j * L + lane
        tok_v[pl.ds(j * L, L)] = jnp.where(v < 0, pvec & (T - 1), v >> 1)
    for c in range(RPW // 64):
        anyreal = jnp.zeros((L,), _i32)
        for j in range(4):
            anyreal = jnp.maximum(anyreal, rf_v[pl.ds(c * 64 + j * L, L)])
        has_real = jnp.max(anyreal) >= 0

        @pl.when(has_real)
        def _():
            cp = pltpu.async_copy(
                hid_hbm.at[tok_v.at[pl.ds(c * 64, 64)]], rows_v, sem)
            cp.wait()
            pltpu.sync_copy(
                rows_v,
                xs_hbm.at[pl.ds(pl.multiple_of(base_r + c * 64, 64), 64)])

    @pl.when(wid == 0)
    def _():
        pltpu.sync_copy(rowflat_hbm, rfall)

        def inv(i, s):
            v = rfall[pl.ds(i * L, L)]
            plsc.store_scatter(posb, [v], i * L + lane, mask=v >= 0)
            return s
        lax.fori_loop(0, NROWS // L, inv, 0)
        pltpu.sync_copy(posb, pos_hbm)


def _gather_rows(hidden, rowflat):
    return pl.kernel(
        _gather_body,
        out_type=(
            jax.ShapeDtypeStruct((NROWS, D), _f32),
            jax.ShapeDtypeStruct((NA,), _i32),
        ),
        mesh=_sc_mesh(),
        compiler_params=pltpu.CompilerParams(needs_layout_passes=False),
        scratch_types=[
            pltpu.VMEM((RPW,), _i32),
            pltpu.VMEM((RPW,), _i32),
            pltpu.VMEM((64, D), _f32),
            pltpu.VMEM((NROWS,), _i32),
            pltpu.VMEM((NA,), _i32),
            pltpu.SemaphoreType.DMA,
        ],
    )(hidden, rowflat)


def _ffn_kernel(seg_ref, x_hbm, w13_hbm, w2_hbm, rw_hbm, o_hbm,
                w13buf, w2buf, xbuf, obuf, rwbuf,
                wsem13, wsem2, xsem, osem, rwsem):
    def w_copies(e, slot):
        return (pltpu.make_async_copy(
                    w13_hbm.at[e], w13buf.at[slot], wsem13.at[slot]),
                pltpu.make_async_copy(
                    w2_hbm.at[e], w2buf.at[slot], wsem2.at[slot]))

    def x_copy(t, slot):
        row = pl.multiple_of(t * TM, TM)
        return pltpu.make_async_copy(
            x_hbm.at[pl.ds(row, TM)], xbuf.at[slot], xsem.at[slot])

    def rw_copy(t, slot):
        row = pl.multiple_of(t * TM, TM)
        return pltpu.make_async_copy(
            rw_hbm.at[pl.ds(row, TM)], rwbuf.at[slot], rwsem.at[slot])

    def o_copy(t, slot):
        row = pl.multiple_of(t * TM, TM)
        return pltpu.make_async_copy(
            obuf.at[slot], o_hbm.at[pl.ds(row, TM)], osem.at[slot])

    for cp in w_copies(0, 0):
        cp.start(priority=1)

    def expert_body(e, carry):
        slot = lax.rem(e, 2)
        t0 = seg_ref[e]
        nt = seg_ref[E + e]

        @pl.when(nt > 0)
        def _():
            x_copy(t0, 0).start()
            rw_copy(t0, 0).start()

        @pl.when(e + 1 < E)
        def _():
            for cp in w_copies(e + 1, 1 - slot):
                cp.start(priority=1)

        for cp in w_copies(e, slot):
            cp.wait()
        w13b = w13buf[slot].astype(jnp.bfloat16)
        w2b = w2buf[slot].astype(jnp.bfloat16)

        @pl.when(nt > 0)
        def _():
            def step(c, s):
                xslot = lax.rem(c, 2)

                @pl.when(c + 1 < nt)
                def _():
                    x_copy(t0 + c + 1, 1 - xslot).start()
                    rw_copy(t0 + c + 1, 1 - xslot).start()

                x_copy(t0 + c, xslot).wait()
                rw_copy(t0 + c, xslot).wait()
                x = xbuf[xslot].astype(jnp.bfloat16)
                gu = jax.lax.dot_general(
                    x, w13b, (((1,), (1,)), ((), ())),
                    preferred_element_type=_f32)
                g = gu[:, :F]
                u = gu[:, F:]
                h = ((g / (1.0 + jnp.exp(-g))) * u).astype(jnp.bfloat16)
                o = jax.lax.dot_general(
                    h, w2b, (((1,), (1,)), ((), ())),
                    preferred_element_type=_f32)

                @pl.when(c >= 2)
                def _():
                    o_copy(t0 + c - 2, xslot).wait()

                obuf[xslot] = o * rwbuf[xslot]
                o_copy(t0 + c, xslot).start()
                return s

            lax.fori_loop(0, nt, step, 0)

            @pl.when(nt >= 2)
            def _():
                o_copy(t0 + nt - 2, lax.rem(nt, 2)).wait()
            o_copy(t0 + nt - 1, lax.rem(nt + 1, 2)).wait()

        return carry

    lax.fori_loop(0, E, expert_body, 0)


def _ffn(seg_info, xsorted, w13, w2, roww):
    grid_spec = pltpu.PrefetchScalarGridSpec(
        num_scalar_prefetch=1,
        grid=(1,),
        in_specs=[
            pl.BlockSpec(memory_space=pl.ANY),
            pl.BlockSpec(memory_space=pl.ANY),
            pl.BlockSpec(memory_space=pl.ANY),
            pl.BlockSpec(memory_space=pl.ANY),
        ],
        out_specs=pl.BlockSpec(memory_space=pl.ANY),
        scratch_shapes=[
            pltpu.VMEM((2, 2 * F, D), _f32),
            pltpu.VMEM((2, D, F), _f32),
            pltpu.VMEM((2, TM, D), _f32),
            pltpu.VMEM((2, TM, D), _f32),
            pltpu.VMEM((2, TM, 1), _f32),
            pltpu.SemaphoreType.DMA((2,)),
            pltpu.SemaphoreType.DMA((2,)),
            pltpu.SemaphoreType.DMA((2,)),
            pltpu.SemaphoreType.DMA((2,)),
            pltpu.SemaphoreType.DMA((2,)),
        ],
    )
    return pl.pallas_call(
        _ffn_kernel,
        grid_spec=grid_spec,
        out_shape=jax.ShapeDtypeStruct((NROWS, D), _f32),
    )(seg_info, xsorted, w13, w2, roww)


def _combine_body(y_hbm, pos_hbm, out_hbm, posv, rows_v, obuf, sem):
    wid = _wid()
    pltpu.sync_copy(
        pos_hbm.at[pl.ds(pl.multiple_of(wid * K * TPW, K * TPW), K * TPW)],
        posv)
    for c in range(K * TPW // 64):
        cp = pltpu.async_copy(
            y_hbm.at[posv.at[pl.ds(c * 64, 64)]], rows_v, sem)
        cp.wait()

        @plsc.parallel_loop(0, 32 * (D // L), unroll=8)
        def addloop(i):
            p = i >> 6
            dd = (i & 63) * L
            a = rows_v[p * 2, pl.ds(dd, L)]
            b = rows_v[p * 2 + 1, pl.ds(dd, L)]
            obuf[p, pl.ds(dd, L)] = a + b

        pltpu.sync_copy(
            obuf, out_hbm.at[pl.ds(pl.multiple_of(wid * TPW + c * 32, 32), 32)])


def _combine(y, pos):
    return pl.kernel(
        _combine_body,
        out_type=jax.ShapeDtypeStruct((T, D), _f32),
        mesh=_sc_mesh(),
        compiler_params=pltpu.CompilerParams(needs_layout_passes=False),
        scratch_types=[
            pltpu.VMEM((K * TPW,), _i32),
            pltpu.VMEM((64, D), _f32),
            pltpu.VMEM((32, D), _f32),
            pltpu.SemaphoreType.DMA,
        ],
    )(y, pos)


@jax.jit
def kernel(hidden_states, gate_weight, w13_weight, w2_weight):
    idx, w, cnt = _router(hidden_states, gate_weight)
    cnt128 = cnt.reshape(NB * E)
    eflat = idx.reshape(NA)
    wflat = w.reshape(NA)
    rowflat, roww, seg_info = _dispatch(cnt128, eflat, wflat)
    xsorted, pos = _gather_rows(hidden_states, rowflat)
    y = _ffn(seg_info, xsorted, w13_weight, w2_weight,
             roww.reshape(NROWS, 1))
    return _combine(y, pos)


# speedup vs baseline: 2.2626x; 1.0169x over previous
import functools

import jax
import jax.numpy as jnp
from jax import lax
from jax.experimental import pallas as pl
from jax.experimental.pallas import tpu as pltpu
from jax.experimental.pallas import tpu_sc as plsc

E = 16
K = 2
D = 1024
F = 768
T = 2048
BT = 256
NB = T // BT
TM = 256
NA = T * K
NROWS = NA + E * TM
NTILES = NROWS // TM
NC = 2
NS = 16
NW = NC * NS
L = 16
RPW = NROWS // NW
TPW = T // NW

_i32 = jnp.int32
_f32 = jnp.float32


def _router_kernel(x_ref, g_ref, idx_ref, w_ref, cnt_ref):
    x = x_ref[...]
    gw = g_ref[...]
    logits = jax.lax.dot_general(
        x.astype(jnp.bfloat16), gw.astype(jnp.bfloat16),
        (((1,), (1,)), ((), ())),
        preferred_element_type=_f32)
    iota_e = jax.lax.broadcasted_iota(_i32, (BT, E), 1)
    m1 = jnp.max(logits, axis=1, keepdims=True)
    i1 = jnp.min(jnp.where(logits == m1, iota_e, E), axis=1, keepdims=True)
    masked = jnp.where(iota_e == i1, -jnp.inf, logits)
    m2 = jnp.max(masked, axis=1, keepdims=True)
    i2 = jnp.min(jnp.where(masked == m2, iota_e, E), axis=1, keepdims=True)
    t = jnp.exp(m2 - m1)
    w1 = 1.0 / (1.0 + t)
    w2v = t / (1.0 + t)
    idx_ref[...] = jnp.concatenate([i1, i2], axis=1)
    w_ref[...] = jnp.concatenate([w1, w2v], axis=1)
    onehot = (iota_e == i1).astype(_i32) + (iota_e == i2).astype(_i32)
    cnt_ref[...] = jnp.sum(onehot, axis=0).reshape(1, 1, E)


def _router(hidden, gate):
    return pl.pallas_call(
        _router_kernel,
        grid=(NB,),
        in_specs=[
            pl.BlockSpec((BT, D), lambda i: (i, 0)),
            pl.BlockSpec((E, D), lambda i: (0, 0)),
        ],
        out_specs=[
            pl.BlockSpec((BT, K), lambda i: (i, 0)),
            pl.BlockSpec((BT, K), lambda i: (i, 0)),
            pl.BlockSpec((1, 1, E), lambda i: (i, 0, 0)),
        ],
        out_shape=[
            jax.ShapeDtypeStruct((T, K), _i32),
            jax.ShapeDtypeStruct((T, K), _f32),
            jax.ShapeDtypeStruct((NB, 1, E), _i32),
        ],
    )(hidden, gate)


def _sc_mesh():
    return plsc.VectorSubcoreMesh(core_axis_name="c", subcore_axis_name="s",
                                  num_cores=NC, num_subcores=NS)


def _wid():
    return lax.axis_index("c") * NS + lax.axis_index("s")


def _lane():
    return lax.broadcasted_iota(_i32, (L,), 0)


def _offsets(cnt_v):
    totals = cnt_v[pl.ds(0, L)]
    for b in range(1, NB):
        totals = totals + cnt_v[pl.ds(b * L, L)]
    padded = ((totals + (TM - 1)) >> 8) << 8
    inc = plsc.cumsum(padded)
    return totals, padded, inc


def _dispatch_body(cnt_hbm, ef_hbm, wf_hbm,
                   rowflat_hbm, roww_hbm, seg_hbm,
                   cnt_v, ef_v, wf_v, flatb, wb, negb, zerob, segb):
    wid = _wid()
    lane = _lane()
    pltpu.sync_copy(cnt_hbm, cnt_v)
    totals, padded, inc = _offsets(cnt_v)
    exc = inc - padded

    @pl.when(wid < E)
    def _():
        e = wid
        base = jnp.sum(jnp.where(lane == e, exc, 0))
        padded_e = jnp.sum(jnp.where(lane == e, padded, 0))
        pltpu.sync_copy(ef_hbm, ef_v)
        pltpu.sync_copy(wf_hbm, wf_v)
        neg1 = jnp.full((L,), -1, _i32)
        zerof = jnp.zeros((L,), _f32)

        def fill(i, c):
            flatb[pl.ds(i * L, L)] = neg1
            wb[pl.ds(i * L, L)] = zerof
            return c
        lax.fori_loop(0, T // L, fill, 0)

        def scan(i, running):
            v = ef_v[pl.ds(i * L, L)]
            m = v == e
            pref = plsc.cumsum(m.astype(_i32))
            dst = running + pref - 1
            plsc.store_scatter(flatb, [dst], i * L + lane, mask=m)
            plsc.store_scatter(wb, [dst], wf_v[pl.ds(i * L, L)], mask=m)
            return running + plsc.all_reduce_population_count(m)
        lax.fori_loop(0, NA // L, scan, jnp.zeros((L,), _i32))

        def dma(c, s):
            off = pl.multiple_of(base + c * TM, TM)
            pltpu.sync_copy(flatb.at[pl.ds(c * TM, TM)],
                            rowflat_hbm.at[pl.ds(off, TM)])
            pltpu.sync_copy(wb.at[pl.ds(c * TM, TM)],
                            roww_hbm.at[pl.ds(off, TM)])
            return s
        lax.fori_loop(0, padded_e >> 8, dma, 0)

    @pl.when(wid == E)
    def _():
        segb[pl.ds(0, L)] = exc >> 8
        segb[pl.ds(L, L)] = padded >> 8
        pltpu.sync_copy(segb, seg_hbm)

    @pl.when(wid == E + 1)
    def _():
        used_tiles = jnp.sum(jnp.where(lane == E - 1, inc, 0)) >> 8
        neg1 = jnp.full((L,), -1, _i32)
        zerof = jnp.zeros((L,), _f32)
        for i in range(TM // L):
            negb[pl.ds(i * L, L)] = neg1
            zerob[pl.ds(i * L, L)] = zerof

        def tail(c, s):
            off = pl.multiple_of(c * TM, TM)
            pltpu.sync_copy(negb, rowflat_hbm.at[pl.ds(off, TM)])
            pltpu.sync_copy(zerob, roww_hbm.at[pl.ds(off, TM)])
            return s
        lax.fori_loop(used_tiles, NTILES, tail, 0)


def _dispatch(cnt128, eflat, wflat):
    return pl.kernel(
        _dispatch_body,
        out_type=(
            jax.ShapeDtypeStruct((NROWS,), _i32),
            jax.ShapeDtypeStruct((NROWS,), _f32),
            jax.ShapeDtypeStruct((2 * E,), _i32),
        ),
        mesh=_sc_mesh(),
        compiler_params=pltpu.CompilerParams(needs_layout_passes=False),
        scratch_types=[
            pltpu.VMEM((NB * E,), _i32),
            pltpu.VMEM((NA,), _i32),
            pltpu.VMEM((NA,), _f32),
            pltpu.VMEM((T,), _i32),
            pltpu.VMEM((T,), _f32),
            pltpu.VMEM((TM,), _i32),
            pltpu.VMEM((TM,), _f32),
            pltpu.VMEM((2 * E,), _i32),
        ],
    )(cnt128, eflat, wflat)


def _gather_body(hid_hbm, rowflat_hbm, xs_hbm, pos_hbm,
                 rf_v, tok_v, rows_v, rfall, posb, sem):
    wid = _wid()
    lane = _lane()
    base_r = pl.multiple_of(wid * RPW, 64)
    pltpu.sync_copy(rowflat_hbm.at[pl.ds(base_r, RPW)], rf_v)
    for j in range(RPW // L):
        v = rf_v[pl.ds(j * L, L)]
        pvec = base_r + j * L + lane
        tok_v[pl.ds(j * L, L)] = jnp.where(v < 0, pvec & (T - 1), v >> 1)
    def chunk_real(c):
        anyreal = jnp.zeros((L,), _i32)
        for j in range(2):
            anyreal = jnp.maximum(anyreal, rf_v[pl.ds(c * 32 + j * L, L)])
        return jnp.max(anyreal) >= 0

    nch = RPW // 32
    reals = [chunk_real(c) for c in range(nch)]

    def chunk_gather(c):
        return pltpu.make_async_copy(
            hid_hbm.at[tok_v.at[pl.ds(c * 32, 32)]],
            rows_v.at[c % 2], sem.at[c % 2])

    @pl.when(reals[0])
    def _():
        chunk_gather(0).start()
    for c in range(nch):
        if c + 1 < nch:
            @pl.when(reals[c + 1])
            def _(c=c):
                chunk_gather(c + 1).start()

        @pl.when(reals[c])
        def _(c=c):
            chunk_gather(c).wait()
            pltpu.sync_copy(
                rows_v.at[c % 2],
                xs_hbm.at[pl.ds(pl.multiple_of(base_r + c * 32, 32), 32)])

    @pl.when(wid == 0)
    def _():
        pltpu.sync_copy(rowflat_hbm, rfall)

        def inv(i, s):
            v = rfall[pl.ds(i * L, L)]
            plsc.store_scatter(posb, [v], i * L + lane, mask=v >= 0)
            return s
        lax.fori_loop(0, NROWS // L, inv, 0)
        pltpu.sync_copy(posb, pos_hbm)


def _gather_rows(hidden, rowflat):
    return pl.kernel(
        _gather_body,
        out_type=(
            jax.ShapeDtypeStruct((NROWS, D), _f32),
            jax.ShapeDtypeStruct((NA,), _i32),
        ),
        mesh=_sc_mesh(),
        compiler_params=pltpu.CompilerParams(needs_layout_passes=False),
        scratch_types=[
            pltpu.VMEM((RPW,), _i32),
            pltpu.VMEM((RPW,), _i32),
            pltpu.VMEM((2, 32, D), _f32),
            pltpu.VMEM((NROWS,), _i32),
            pltpu.VMEM((NA,), _i32),
            pltpu.SemaphoreType.DMA((2,)),
        ],
    )(hidden, rowflat)


def _ffn_kernel(seg_ref, x_hbm, w13_hbm, w2_hbm, rw_hbm, o_hbm,
                w13buf, w2buf, xbuf, obuf, rwbuf,
                wsem13, wsem2, xsem, osem, rwsem):
    def w_copies(e, slot):
        return (pltpu.make_async_copy(
                    w13_hbm.at[e], w13buf.at[slot], wsem13.at[slot]),
                pltpu.make_async_copy(
                    w2_hbm.at[e], w2buf.at[slot], wsem2.at[slot]))

    def x_copy(t, slot):
        row = pl.multiple_of(t * TM, TM)
        return pltpu.make_async_copy(
            x_hbm.at[pl.ds(row, TM)], xbuf.at[slot], xsem.at[slot])

    def rw_copy(t, slot):
        row = pl.multiple_of(t * TM, TM)
        return pltpu.make_async_copy(
            rw_hbm.at[pl.ds(row, TM)], rwbuf.at[slot], rwsem.at[slot])

    def o_copy(t, slot):
        row = pl.multiple_of(t * TM, TM)
        return pltpu.make_async_copy(
            obuf.at[slot], o_hbm.at[pl.ds(row, TM)], osem.at[slot])

    for cp in w_copies(0, 0):
        cp.start(priority=1)

    def expert_body(e, carry):
        slot = lax.rem(e, 2)
        t0 = seg_ref[e]
        nt = seg_ref[E + e]

        @pl.when(nt > 0)
        def _():
            x_copy(t0, 0).start()
            rw_copy(t0, 0).start()

        @pl.when(e + 1 < E)
        def _():
            for cp in w_copies(e + 1, 1 - slot):
                cp.start(priority=1)

        for cp in w_copies(e, slot):
            cp.wait()
        w13b = w13buf[slot].astype(jnp.bfloat16)
        w2b = w2buf[slot].astype(jnp.bfloat16)

        @pl.when(nt > 0)
        def _():
            def step(c, s):
                xslot = lax.rem(c, 2)

                @pl.when(c + 1 < nt)
                def _():
                    x_copy(t0 + c + 1, 1 - xslot).start()
                    rw_copy(t0 + c + 1, 1 - xslot).start()

                x_copy(t0 + c, xslot).wait()
                rw_copy(t0 + c, xslot).wait()
                x = xbuf[xslot].astype(jnp.bfloat16)
                gu = jax.lax.dot_general(
                    x, w13b, (((1,), (1,)), ((), ())),
                    preferred_element_type=_f32)
                g = gu[:, :F]
                u = gu[:, F:]
                h = ((g / (1.0 + jnp.exp(-g))) * u).astype(jnp.bfloat16)
                o = jax.lax.dot_general(
                    h, w2b, (((1,), (1,)), ((), ())),
                    preferred_element_type=_f32)

                @pl.when(c >= 2)
                def _():
                    o_copy(t0 + c - 2, xslot).wait()

                obuf[xslot] = o * rwbuf[xslot]
                o_copy(t0 + c, xslot).start()
                return s

            lax.fori_loop(0, nt, step, 0)

            @pl.when(nt >= 2)
            def _():
                o_copy(t0 + nt - 2, lax.rem(nt, 2)).wait()
            o_copy(t0 + nt - 1, lax.rem(nt + 1, 2)).wait()

        return carry

    lax.fori_loop(0, E, expert_body, 0)


def _ffn(seg_info, xsorted, w13, w2, roww):
    grid_spec = pltpu.PrefetchScalarGridSpec(
        num_scalar_prefetch=1,
        grid=(1,),
        in_specs=[
            pl.BlockSpec(memory_space=pl.ANY),
            pl.BlockSpec(memory_space=pl.ANY),
            pl.BlockSpec(memory_space=pl.ANY),
            pl.BlockSpec(memory_space=pl.ANY),
        ],
        out_specs=pl.BlockSpec(memory_space=pl.ANY),
        scratch_shapes=[
            pltpu.VMEM((2, 2 * F, D), _f32),
            pltpu.VMEM((2, D, F), _f32),
            pltpu.VMEM((2, TM, D), _f32),
            pltpu.VMEM((2, TM, D), _f32),
            pltpu.VMEM((2, TM, 1), _f32),
            pltpu.SemaphoreType.DMA((2,)),
            pltpu.SemaphoreType.DMA((2,)),
            pltpu.SemaphoreType.DMA((2,)),
            pltpu.SemaphoreType.DMA((2,)),
            pltpu.SemaphoreType.DMA((2,)),
        ],
    )
    return pl.pallas_call(
        _ffn_kernel,
        grid_spec=grid_spec,
        out_shape=jax.ShapeDtypeStruct((NROWS, D), _f32),
    )(seg_info, xsorted, w13, w2, roww)


def _combine_body(y_hbm, pos_hbm, out_hbm, posv, rows_v, obuf, sem):
    wid = _wid()
    pltpu.sync_copy(
        pos_hbm.at[pl.ds(pl.multiple_of(wid * K * TPW, K * TPW), K * TPW)],
        posv)
    for c in range(K * TPW // 64):
        cp = pltpu.async_copy(
            y_hbm.at[posv.at[pl.ds(c * 64, 64)]], rows_v, sem)
        cp.wait()

        @plsc.parallel_loop(0, 32 * (D // L), unroll=8)
        def addloop(i):
            p = i >> 6
            dd = (i & 63) * L
            a = rows_v[p * 2, pl.ds(dd, L)]
            b = rows_v[p * 2 + 1, pl.ds(dd, L)]
            obuf[p, pl.ds(dd, L)] = a + b

        pltpu.sync_copy(
            obuf, out_hbm.at[pl.ds(pl.multiple_of(wid * TPW + c * 32, 32), 32)])


def _combine(y, pos):
    return pl.kernel(
        _combine_body,
        out_type=jax.ShapeDtypeStruct((T, D), _f32),
        mesh=_sc_mesh(),
        compiler_params=pltpu.CompilerParams(needs_layout_passes=False),
        scratch_types=[
            pltpu.VMEM((K * TPW,), _i32),
            pltpu.VMEM((64, D), _f32),
            pltpu.VMEM((32, D), _f32),
            pltpu.SemaphoreType.DMA,
        ],
    )(y, pos)


@jax.jit
def kernel(hidden_states, gate_weight, w13_weight, w2_weight):
    idx, w, cnt = _router(hidden_states, gate_weight)
    cnt128 = cnt.reshape(NB * E)
    eflat = idx.reshape(NA)
    wflat = w.reshape(NA)
    rowflat, roww, seg_info = _dispatch(cnt128, eflat, wflat)
    xsorted, pos = _gather_rows(hidden_states, rowflat)
    y = _ffn(seg_info, xsorted, w13_weight, w2_weight,
             roww.reshape(NROWS, 1))
    return _combine(y, pos)


# speedup vs baseline: 2.2945x; 1.0141x over previous
import functools

import jax
import jax.numpy as jnp
from jax import lax
from jax.experimental import pallas as pl
from jax.experimental.pallas import tpu as pltpu
from jax.experimental.pallas import tpu_sc as plsc

E = 16
K = 2
D = 1024
F = 768
T = 2048
BT = 256
NB = T // BT
TM = 256
NA = T * K
NROWS = NA + E * TM
NTILES = NROWS // TM
NC = 2
NS = 16
NW = NC * NS
L = 16
RPW = NROWS // NW
TPW = T // NW

_i32 = jnp.int32
_f32 = jnp.float32


def _router_kernel(x_ref, g_ref, idx_ref, w_ref, cnt_ref):
    x = x_ref[...]
    gw = g_ref[...]
    logits = jax.lax.dot_general(
        x.astype(jnp.bfloat16), gw.astype(jnp.bfloat16),
        (((1,), (1,)), ((), ())),
        preferred_element_type=_f32)
    iota_e = jax.lax.broadcasted_iota(_i32, (BT, E), 1)
    m1 = jnp.max(logits, axis=1, keepdims=True)
    i1 = jnp.min(jnp.where(logits == m1, iota_e, E), axis=1, keepdims=True)
    masked = jnp.where(iota_e == i1, -jnp.inf, logits)
    m2 = jnp.max(masked, axis=1, keepdims=True)
    i2 = jnp.min(jnp.where(masked == m2, iota_e, E), axis=1, keepdims=True)
    t = jnp.exp(m2 - m1)
    w1 = 1.0 / (1.0 + t)
    w2v = t / (1.0 + t)
    idx_ref[...] = jnp.concatenate([i1, i2], axis=1)
    w_ref[...] = jnp.concatenate([w1, w2v], axis=1)
    onehot = (iota_e == i1).astype(_i32) + (iota_e == i2).astype(_i32)
    cnt_ref[...] = jnp.sum(onehot, axis=0).reshape(1, 1, E)


def _router(hidden, gate):
    return pl.pallas_call(
        _router_kernel,
        grid=(NB,),
        in_specs=[
            pl.BlockSpec((BT, D), lambda i: (i, 0)),
            pl.BlockSpec((E, D), lambda i: (0, 0)),
        ],
        out_specs=[
            pl.BlockSpec((BT, K), lambda i: (i, 0)),
            pl.BlockSpec((BT, K), lambda i: (i, 0)),
            pl.BlockSpec((1, 1, E), lambda i: (i, 0, 0)),
        ],
        out_shape=[
            jax.ShapeDtypeStruct((T, K), _i32),
            jax.ShapeDtypeStruct((T, K), _f32),
            jax.ShapeDtypeStruct((NB, 1, E), _i32),
        ],
    )(hidden, gate)


def _sc_mesh():
    return plsc.VectorSubcoreMesh(core_axis_name="c", subcore_axis_name="s",
                                  num_cores=NC, num_subcores=NS)


def _wid():
    return lax.axis_index("c") * NS + lax.axis_index("s")


def _lane():
    return lax.broadcasted_iota(_i32, (L,), 0)


def _offsets(cnt_v):
    totals = cnt_v[pl.ds(0, L)]
    for b in range(1, NB):
        totals = totals + cnt_v[pl.ds(b * L, L)]
    padded = ((totals + (TM - 1)) >> 8) << 8
    inc = plsc.cumsum(padded)
    return totals, padded, inc


def _dispatch_body(cnt_hbm, ef_hbm, wf_hbm,
                   rowflat_hbm, roww_hbm, seg_hbm,
                   cnt_v, ef_v, wf_v, flatb, wb, negb, zerob, segb):
    wid = _wid()
    lane = _lane()
    pltpu.sync_copy(cnt_hbm, cnt_v)
    totals, padded, inc = _offsets(cnt_v)
    exc = inc - padded

    @pl.when(wid < E)
    def _():
        e = wid
        base = jnp.sum(jnp.where(lane == e, exc, 0))
        padded_e = jnp.sum(jnp.where(lane == e, padded, 0))
        pltpu.sync_copy(ef_hbm, ef_v)
        pltpu.sync_copy(wf_hbm, wf_v)
        neg1 = jnp.full((L,), -1, _i32)
        zerof = jnp.zeros((L,), _f32)

        def fill(i, c):
            flatb[pl.ds(i * L, L)] = neg1
            wb[pl.ds(i * L, L)] = zerof
            return c
        lax.fori_loop(0, T // L, fill, 0)

        def scan(i, running):
            v = ef_v[pl.ds(i * L, L)]
            m = v == e
            pref = plsc.cumsum(m.astype(_i32))
            dst = running + pref - 1
            plsc.store_scatter(flatb, [dst], i * L + lane, mask=m)
            plsc.store_scatter(wb, [dst], wf_v[pl.ds(i * L, L)], mask=m)
            return running + plsc.all_reduce_population_count(m)
        lax.fori_loop(0, NA // L, scan, jnp.zeros((L,), _i32))

        def dma(c, s):
            off = pl.multiple_of(base + c * TM, TM)
            pltpu.sync_copy(flatb.at[pl.ds(c * TM, TM)],
                            rowflat_hbm.at[pl.ds(off, TM)])
            pltpu.sync_copy(wb.at[pl.ds(c * TM, TM)],
                            roww_hbm.at[pl.ds(off, TM)])
            return s
        lax.fori_loop(0, padded_e >> 8, dma, 0)

    @pl.when(wid == E)
    def _():
        segb[pl.ds(0, L)] = exc >> 8
        segb[pl.ds(L, L)] = padded >> 8
        pltpu.sync_copy(segb, seg_hbm)

    @pl.when(wid == E + 1)
    def _():
        used_tiles = jnp.sum(jnp.where(lane == E - 1, inc, 0)) >> 8
        neg1 = jnp.full((L,), -1, _i32)
        zerof = jnp.zeros((L,), _f32)
        for i in range(TM // L):
            negb[pl.ds(i * L, L)] = neg1
            zerob[pl.ds(i * L, L)] = zerof

        def tail(c, s):
            off = pl.multiple_of(c * TM, TM)
            pltpu.sync_copy(negb, rowflat_hbm.at[pl.ds(off, TM)])
            pltpu.sync_copy(zerob, roww_hbm.at[pl.ds(off, TM)])
            return s
        lax.fori_loop(used_tiles, NTILES, tail, 0)


def _dispatch(cnt128, eflat, wflat):
    return pl.kernel(
        _dispatch_body,
        out_type=(
            jax.ShapeDtypeStruct((NROWS,), _i32),
            jax.ShapeDtypeStruct((NROWS,), _f32),
            jax.ShapeDtypeStruct((2 * E,), _i32),
        ),
        mesh=_sc_mesh(),
        compiler_params=pltpu.CompilerParams(needs_layout_passes=False),
        scratch_types=[
            pltpu.VMEM((NB * E,), _i32),
            pltpu.VMEM((NA,), _i32),
            pltpu.VMEM((NA,), _f32),
            pltpu.VMEM((T,), _i32),
            pltpu.VMEM((T,), _f32),
            pltpu.VMEM((TM,), _i32),
            pltpu.VMEM((TM,), _f32),
            pltpu.VMEM((2 * E,), _i32),
        ],
    )(cnt128, eflat, wflat)


def _gather_body(hid_hbm, rowflat_hbm, xs_hbm, pos_hbm,
                 rf_v, tok_v, rows_v, rfall, posb, sem):
    wid = _wid()
    lane = _lane()
    base_r = pl.multiple_of(wid * RPW, 64)
    pltpu.sync_copy(rowflat_hbm.at[pl.ds(base_r, RPW)], rf_v)
    for j in range(RPW // L):
        v = rf_v[pl.ds(j * L, L)]
        pvec = base_r + j * L + lane
        tok_v[pl.ds(j * L, L)] = jnp.where(v < 0, pvec & (T - 1), v >> 1)
    def chunk_real(c):
        anyreal = jnp.zeros((L,), _i32)
        for j in range(2):
            anyreal = jnp.maximum(anyreal, rf_v[pl.ds(c * 32 + j * L, L)])
        return jnp.max(anyreal) >= 0

    nch = RPW // 32
    reals = [chunk_real(c) for c in range(nch)]

    def chunk_gather(c):
        return pltpu.make_async_copy(
            hid_hbm.at[tok_v.at[pl.ds(c * 32, 32)]],
            rows_v.at[c % 2], sem.at[c % 2])

    @pl.when(reals[0])
    def _():
        chunk_gather(0).start()
    for c in range(nch):
        if c + 1 < nch:
            @pl.when(reals[c + 1])
            def _(c=c):
                chunk_gather(c + 1).start()

        @pl.when(reals[c])
        def _(c=c):
            chunk_gather(c).wait()
            pltpu.sync_copy(
                rows_v.at[c % 2],
                xs_hbm.at[pl.ds(pl.multiple_of(base_r + c * 32, 32), 32)])

    @pl.when(wid == 0)
    def _():
        pltpu.sync_copy(rowflat_hbm, rfall)

        def inv(i, s):
            v = rfall[pl.ds(i * L, L)]
            plsc.store_scatter(posb, [v], i * L + lane, mask=v >= 0)
            return s
        lax.fori_loop(0, NROWS // L, inv, 0)
        pltpu.sync_copy(posb, pos_hbm)


def _gather_rows(hidden, rowflat):
    return pl.kernel(
        _gather_body,
        out_type=(
            jax.ShapeDtypeStruct((NROWS, D), _f32),
            jax.ShapeDtypeStruct((NA,), _i32),
        ),
        mesh=_sc_mesh(),
        compiler_params=pltpu.CompilerParams(needs_layout_passes=False),
        scratch_types=[
            pltpu.VMEM((RPW,), _i32),
            pltpu.VMEM((RPW,), _i32),
            pltpu.VMEM((2, 32, D), _f32),
            pltpu.VMEM((NROWS,), _i32),
            pltpu.VMEM((NA,), _i32),
            pltpu.SemaphoreType.DMA((2,)),
        ],
    )(hidden, rowflat)


def _ffn_kernel(seg_ref, x_hbm, w13_hbm, w2_hbm, rw_hbm, o_hbm,
                w13buf, w2buf, xbuf, obuf, rwbuf,
                wsem13, wsem2, xsem, osem, rwsem):
    def w_copies(e, slot):
        return (pltpu.make_async_copy(
                    w13_hbm.at[e], w13buf.at[slot], wsem13.at[slot]),
                pltpu.make_async_copy(
                    w2_hbm.at[e], w2buf.at[slot], wsem2.at[slot]))

    def x_copy(t, slot):
        row = pl.multiple_of(t * TM, TM)
        return pltpu.make_async_copy(
            x_hbm.at[pl.ds(row, TM)], xbuf.at[slot], xsem.at[slot])

    def rw_copy(t, slot):
        row = pl.multiple_of(t * TM, TM)
        return pltpu.make_async_copy(
            rw_hbm.at[pl.ds(row, TM)], rwbuf.at[slot], rwsem.at[slot])

    def o_copy(t, slot):
        row = pl.multiple_of(t * TM, TM)
        return pltpu.make_async_copy(
            obuf.at[slot], o_hbm.at[pl.ds(row, TM)], osem.at[slot])

    for cp in w_copies(0, 0):
        cp.start(priority=1)

    def expert_body(e, carry):
        slot = lax.rem(e, 2)
        t0 = seg_ref[e]
        nt = seg_ref[E + e]

        @pl.when(nt > 0)
        def _():
            x_copy(t0, 0).start()
            rw_copy(t0, 0).start()

        @pl.when(e + 1 < E)
        def _():
            for cp in w_copies(e + 1, 1 - slot):
                cp.start(priority=1)

        for cp in w_copies(e, slot):
            cp.wait()
        w13b = w13buf[slot].astype(jnp.bfloat16)
        w2b = w2buf[slot].astype(jnp.bfloat16)

        @pl.when(nt > 0)
        def _():
            def step(c, s):
                xslot = lax.rem(c, 2)

                @pl.when(c + 1 < nt)
                def _():
                    x_copy(t0 + c + 1, 1 - xslot).start()
                    rw_copy(t0 + c + 1, 1 - xslot).start()

                x_copy(t0 + c, xslot).wait()
                rw_copy(t0 + c, xslot).wait()
                x = xbuf[xslot].astype(jnp.bfloat16)
                gu = jax.lax.dot_general(
                    x, w13b, (((1,), (1,)), ((), ())),
                    preferred_element_type=_f32)
                g = gu[:, :F]
                u = gu[:, F:]
                h = ((g / (1.0 + jnp.exp(-g))) * u).astype(jnp.bfloat16)
                o = jax.lax.dot_general(
                    h, w2b, (((1,), (1,)), ((), ())),
                    preferred_element_type=_f32)

                @pl.when(c >= 2)
                def _():
                    o_copy(t0 + c - 2, xslot).wait()

                obuf[xslot] = o * rwbuf[xslot]
                o_copy(t0 + c, xslot).start()
                return s

            lax.fori_loop(0, nt, step, 0)

            @pl.when(nt >= 2)
            def _():
                o_copy(t0 + nt - 2, lax.rem(nt, 2)).wait()
            o_copy(t0 + nt - 1, lax.rem(nt + 1, 2)).wait()

        return carry

    lax.fori_loop(0, E, expert_body, 0)


def _ffn(seg_info, xsorted, w13, w2, roww):
    grid_spec = pltpu.PrefetchScalarGridSpec(
        num_scalar_prefetch=1,
        grid=(1,),
        in_specs=[
            pl.BlockSpec(memory_space=pl.ANY),
            pl.BlockSpec(memory_space=pl.ANY),
            pl.BlockSpec(memory_space=pl.ANY),
            pl.BlockSpec(memory_space=pl.ANY),
        ],
        out_specs=pl.BlockSpec(memory_space=pl.ANY),
        scratch_shapes=[
            pltpu.VMEM((2, 2 * F, D), _f32),
            pltpu.VMEM((2, D, F), _f32),
            pltpu.VMEM((2, TM, D), _f32),
            pltpu.VMEM((2, TM, D), _f32),
            pltpu.VMEM((2, TM, 1), _f32),
            pltpu.SemaphoreType.DMA((2,)),
            pltpu.SemaphoreType.DMA((2,)),
            pltpu.SemaphoreType.DMA((2,)),
            pltpu.SemaphoreType.DMA((2,)),
            pltpu.SemaphoreType.DMA((2,)),
        ],
    )
    return pl.pallas_call(
        _ffn_kernel,
        grid_spec=grid_spec,
        out_shape=jax.ShapeDtypeStruct((NROWS, D), _f32),
    )(seg_info, xsorted, w13, w2, roww)


def _combine_body(y_hbm, pos_hbm, out_hbm, posv, rows_v, obuf, sem):
    wid = _wid()
    pltpu.sync_copy(
        pos_hbm.at[pl.ds(pl.multiple_of(wid * K * TPW, K * TPW), K * TPW)],
        posv)
    nch = K * TPW // 32

    def chunk_gather(c):
        return pltpu.make_async_copy(
            y_hbm.at[posv.at[pl.ds(c * 32, 32)]],
            rows_v.at[c % 2], sem.at[c % 2])

    chunk_gather(0).start()
    for c in range(nch):
        if c + 1 < nch:
            chunk_gather(c + 1).start()
        chunk_gather(c).wait()
        rv = rows_v.at[c % 2]

        @plsc.parallel_loop(0, 16 * (D // L), unroll=8)
        def addloop(i):
            p = i >> 6
            dd = (i & 63) * L
            a = rv[p * 2, pl.ds(dd, L)]
            b = rv[p * 2 + 1, pl.ds(dd, L)]
            obuf[p, pl.ds(dd, L)] = a + b

        pltpu.sync_copy(
            obuf, out_hbm.at[pl.ds(pl.multiple_of(wid * TPW + c * 16, 16), 16)])


def _combine(y, pos):
    return pl.kernel(
        _combine_body,
        out_type=jax.ShapeDtypeStruct((T, D), _f32),
        mesh=_sc_mesh(),
        compiler_params=pltpu.CompilerParams(needs_layout_passes=False),
        scratch_types=[
            pltpu.VMEM((K * TPW,), _i32),
            pltpu.VMEM((2, 32, D), _f32),
            pltpu.VMEM((16, D), _f32),
            pltpu.SemaphoreType.DMA((2,)),
        ],
    )(y, pos)


@jax.jit
def kernel(hidden_states, gate_weight, w13_weight, w2_weight):
    idx, w, cnt = _router(hidden_states, gate_weight)
    cnt128 = cnt.reshape(NB * E)
    eflat = idx.reshape(NA)
    wflat = w.reshape(NA)
    rowflat, roww, seg_info = _dispatch(cnt128, eflat, wflat)
    xsorted, pos = _gather_rows(hidden_states, rowflat)
    y = _ffn(seg_info, xsorted, w13_weight, w2_weight,
             roww.reshape(NROWS, 1))
    return _combine(y, pos)


# speedup vs baseline: 2.3872x; 1.0404x over previous
import functools

import jax
import jax.numpy as jnp
from jax import lax
from jax.experimental import pallas as pl
from jax.experimental.pallas import tpu as pltpu
from jax.experimental.pallas import tpu_sc as plsc

E = 16
K = 2
D = 1024
F = 768
T = 2048
BT = 256
NB = T // BT
TM = 256
NA = T * K
NROWS = NA + E * TM
NTILES = NROWS // TM
NC = 2
NS = 16
NW = NC * NS
L = 16
RPW = NROWS // NW
TPW = T // NW

_i32 = jnp.int32
_f32 = jnp.float32


def _router_kernel(x_ref, g_ref, idx_ref, w_ref, cnt_ref):
    x = x_ref[...]
    gw = g_ref[...]
    logits = jax.lax.dot_general(
        x.astype(jnp.bfloat16), gw.astype(jnp.bfloat16),
        (((1,), (1,)), ((), ())),
        preferred_element_type=_f32)
    iota_e = jax.lax.broadcasted_iota(_i32, (BT, E), 1)
    m1 = jnp.max(logits, axis=1, keepdims=True)
    i1 = jnp.min(jnp.where(logits == m1, iota_e, E), axis=1, keepdims=True)
    masked = jnp.where(iota_e == i1, -jnp.inf, logits)
    m2 = jnp.max(masked, axis=1, keepdims=True)
    i2 = jnp.min(jnp.where(masked == m2, iota_e, E), axis=1, keepdims=True)
    t = jnp.exp(m2 - m1)
    w1 = 1.0 / (1.0 + t)
    w2v = t / (1.0 + t)
    idx_ref[...] = jnp.concatenate([i1, i2], axis=1)
    w_ref[...] = jnp.concatenate([w1, w2v], axis=1)
    onehot = (iota_e == i1).astype(_i32) + (iota_e == i2).astype(_i32)
    cnt_ref[...] = jnp.sum(onehot, axis=0).reshape(1, 1, E)


def _router(hidden, gate):
    return pl.pallas_call(
        _router_kernel,
        grid=(NB,),
        in_specs=[
            pl.BlockSpec((BT, D), lambda i: (i, 0)),
            pl.BlockSpec((E, D), lambda i: (0, 0)),
        ],
        out_specs=[
            pl.BlockSpec((BT, K), lambda i: (i, 0)),
            pl.BlockSpec((BT, K), lambda i: (i, 0)),
            pl.BlockSpec((1, 1, E), lambda i: (i, 0, 0)),
        ],
        out_shape=[
            jax.ShapeDtypeStruct((T, K), _i32),
            jax.ShapeDtypeStruct((T, K), _f32),
            jax.ShapeDtypeStruct((NB, 1, E), _i32),
        ],
    )(hidden, gate)


def _sc_mesh():
    return plsc.VectorSubcoreMesh(core_axis_name="c", subcore_axis_name="s",
                                  num_cores=NC, num_subcores=NS)


def _wid():
    return lax.axis_index("c") * NS + lax.axis_index("s")


def _lane():
    return lax.broadcasted_iota(_i32, (L,), 0)


def _offsets(cnt_v):
    totals = cnt_v[pl.ds(0, L)]
    for b in range(1, NB):
        totals = totals + cnt_v[pl.ds(b * L, L)]
    padded = ((totals + (TM - 1)) >> 8) << 8
    inc = plsc.cumsum(padded)
    return totals, padded, inc


def _dispatch_body(cnt_hbm, ef_hbm, wf_hbm,
                   rowflat_hbm, roww_hbm, seg_hbm,
                   cnt_v, ef_v, wf_v, flatb, wb, negb, zerob, segb):
    wid = _wid()
    lane = _lane()
    pltpu.sync_copy(cnt_hbm, cnt_v)
    totals, padded, inc = _offsets(cnt_v)
    exc = inc - padded

    @pl.when(wid < E)
    def _():
        e = wid
        base = jnp.sum(jnp.where(lane == e, exc, 0))
        padded_e = jnp.sum(jnp.where(lane == e, padded, 0))
        pltpu.sync_copy(ef_hbm, ef_v)
        pltpu.sync_copy(wf_hbm, wf_v)
        neg1 = jnp.full((L,), -1, _i32)
        zerof = jnp.zeros((L,), _f32)

        def fill(i, c):
            flatb[pl.ds(i * L, L)] = neg1
            wb[pl.ds(i * L, L)] = zerof
            return c
        lax.fori_loop(0, T // L, fill, 0)

        def scan(i, running):
            v = ef_v[pl.ds(i * L, L)]
            m = v == e
            pref = plsc.cumsum(m.astype(_i32))
            dst = running + pref - 1
            plsc.store_scatter(flatb, [dst], i * L + lane, mask=m)
            plsc.store_scatter(wb, [dst], wf_v[pl.ds(i * L, L)], mask=m)
            return running + plsc.all_reduce_population_count(m)
        lax.fori_loop(0, NA // L, scan, jnp.zeros((L,), _i32))

        def dma(c, s):
            off = pl.multiple_of(base + c * TM, TM)
            pltpu.sync_copy(flatb.at[pl.ds(c * TM, TM)],
                            rowflat_hbm.at[pl.ds(off, TM)])
            pltpu.sync_copy(wb.at[pl.ds(c * TM, TM)],
                            roww_hbm.at[pl.ds(off, TM)])
            return s
        lax.fori_loop(0, padded_e >> 8, dma, 0)

    @pl.when(wid == E)
    def _():
        segb[pl.ds(0, L)] = exc >> 8
        segb[pl.ds(L, L)] = padded >> 8
        pltpu.sync_copy(segb, seg_hbm)

    @pl.when(wid == E + 1)
    def _():
        used_tiles = jnp.sum(jnp.where(lane == E - 1, inc, 0)) >> 8
        neg1 = jnp.full((L,), -1, _i32)
        zerof = jnp.zeros((L,), _f32)
        for i in range(TM // L):
            negb[pl.ds(i * L, L)] = neg1
            zerob[pl.ds(i * L, L)] = zerof

        def tail(c, s):
            off = pl.multiple_of(c * TM, TM)
            pltpu.sync_copy(negb, rowflat_hbm.at[pl.ds(off, TM)])
            pltpu.sync_copy(zerob, roww_hbm.at[pl.ds(off, TM)])
            return s
        lax.fori_loop(used_tiles, NTILES, tail, 0)


def _dispatch(cnt128, eflat, wflat):
    return pl.kernel(
        _dispatch_body,
        out_type=(
            jax.ShapeDtypeStruct((NROWS,), _i32),
            jax.ShapeDtypeStruct((NROWS,), _f32),
            jax.ShapeDtypeStruct((2 * E,), _i32),
        ),
        mesh=_sc_mesh(),
        compiler_params=pltpu.CompilerParams(needs_layout_passes=False),
        scratch_types=[
            pltpu.VMEM((NB * E,), _i32),
            pltpu.VMEM((NA,), _i32),
            pltpu.VMEM((NA,), _f32),
            pltpu.VMEM((T,), _i32),
            pltpu.VMEM((T,), _f32),
            pltpu.VMEM((TM,), _i32),
            pltpu.VMEM((TM,), _f32),
            pltpu.VMEM((2 * E,), _i32),
        ],
    )(cnt128, eflat, wflat)


def _gather_body(hid_hbm, rowflat_hbm, xs_hbm, pos_hbm,
                 rf_v, tok_v, rows_v, rfall, posb, sem):
    wid = _wid()
    lane = _lane()
    base_r = pl.multiple_of(wid * RPW, 64)
    pltpu.sync_copy(rowflat_hbm.at[pl.ds(base_r, RPW)], rf_v)
    for j in range(RPW // L):
        v = rf_v[pl.ds(j * L, L)]
        pvec = base_r + j * L + lane
        tok_v[pl.ds(j * L, L)] = jnp.where(v < 0, pvec & (T - 1), v >> 1)
    def chunk_real(c):
        anyreal = jnp.zeros((L,), _i32)
        for j in range(2):
            anyreal = jnp.maximum(anyreal, rf_v[pl.ds(c * 32 + j * L, L)])
        return jnp.max(anyreal) >= 0

    nch = RPW // 32
    reals = [chunk_real(c) for c in range(nch)]

    def chunk_gather(c):
        return pltpu.make_async_copy(
            hid_hbm.at[tok_v.at[pl.ds(c * 32, 32)]],
            rows_v.at[c % 2], sem.at[c % 2])

    @pl.when(reals[0])
    def _():
        chunk_gather(0).start()
    for c in range(nch):
        if c + 1 < nch:
            @pl.when(reals[c + 1])
            def _(c=c):
                chunk_gather(c + 1).start()

        @pl.when(reals[c])
        def _(c=c):
            chunk_gather(c).wait()
            pltpu.sync_copy(
                rows_v.at[c % 2],
                xs_hbm.at[pl.ds(pl.multiple_of(base_r + c * 32, 32), 32)])

    @pl.when(wid == 0)
    def _():
        pltpu.sync_copy(rowflat_hbm, rfall)

        def inv(i, s):
            v = rfall[pl.ds(i * L, L)]
            plsc.store_scatter(posb, [v], i * L + lane, mask=v >= 0)
            return s
        lax.fori_loop(0, NROWS // L, inv, 0)
        pltpu.sync_copy(posb, pos_hbm)


def _gather_rows(hidden, rowflat):
    return pl.kernel(
        _gather_body,
        out_type=(
            jax.ShapeDtypeStruct((NROWS, D), _f32),
            jax.ShapeDtypeStruct((NA,), _i32),
        ),
        mesh=_sc_mesh(),
        compiler_params=pltpu.CompilerParams(needs_layout_passes=False),
        scratch_types=[
            pltpu.VMEM((RPW,), _i32),
            pltpu.VMEM((RPW,), _i32),
            pltpu.VMEM((2, 32, D), _f32),
            pltpu.VMEM((NROWS,), _i32),
            pltpu.VMEM((NA,), _i32),
            pltpu.SemaphoreType.DMA((2,)),
        ],
    )(hidden, rowflat)


def _ffn_kernel(seg_ref, x_hbm, w13_hbm, w2_hbm, rw_hbm, o_hbm,
                w13buf, w2buf, xbuf, obuf, rwbuf,
                wsem13, wsem2, xsem, osem, rwsem):
    def w_copies(e, slot):
        return (pltpu.make_async_copy(
                    w13_hbm.at[e], w13buf.at[slot], wsem13.at[slot]),
                pltpu.make_async_copy(
                    w2_hbm.at[e], w2buf.at[slot], wsem2.at[slot]))

    def x_copy(t, slot):
        row = pl.multiple_of(t * TM, TM)
        return pltpu.make_async_copy(
            x_hbm.at[pl.ds(row, TM)], xbuf.at[slot], xsem.at[slot])

    def rw_copy(t, slot):
        row = pl.multiple_of(t * TM, TM)
        return pltpu.make_async_copy(
            rw_hbm.at[pl.ds(row, TM)], rwbuf.at[slot], rwsem.at[slot])

    def o_copy(t, slot):
        row = pl.multiple_of(t * TM, TM)
        return pltpu.make_async_copy(
            obuf.at[slot], o_hbm.at[pl.ds(row, TM)], osem.at[slot])

    for cp in w_copies(0, 0):
        cp.start(priority=1)

    ut = seg_ref[E - 1] + seg_ref[2 * E - 1]

    @pl.when(ut > 0)
    def _():
        x_copy(0, 0).start()
        rw_copy(0, 0).start()

    @pl.when(ut > 1)
    def _():
        x_copy(1, 1).start()
        rw_copy(1, 1).start()

    def expert_body(e, carry):
        slot = lax.rem(e, 2)
        t0 = seg_ref[e]
        nt = seg_ref[E + e]

        @pl.when(e + 1 < E)
        def _():
            for cp in w_copies(e + 1, 1 - slot):
                cp.start(priority=1)

        for cp in w_copies(e, slot):
            cp.wait()
        w13b = w13buf[slot].astype(jnp.bfloat16)
        w2b = w2buf[slot].astype(jnp.bfloat16)

        @pl.when(nt > 0)
        def _():
            def step(c, s):
                gt = t0 + c
                xslot = lax.rem(gt, 3)

                @pl.when(gt + 2 < ut)
                def _():
                    x_copy(gt + 2, lax.rem(gt + 2, 3)).start()
                    rw_copy(gt + 2, lax.rem(gt + 2, 3)).start()

                x_copy(gt, xslot).wait()
                rw_copy(gt, xslot).wait()
                x = xbuf[xslot].astype(jnp.bfloat16)
                gu = jax.lax.dot_general(
                    x, w13b, (((1,), (1,)), ((), ())),
                    preferred_element_type=_f32)
                g = gu[:, :F]
                u = gu[:, F:]
                h = ((g / (1.0 + jnp.exp(-g))) * u).astype(jnp.bfloat16)
                o = jax.lax.dot_general(
                    h, w2b, (((1,), (1,)), ((), ())),
                    preferred_element_type=_f32)

                oslot = lax.rem(c, 2)

                @pl.when(c >= 2)
                def _():
                    o_copy(t0 + c - 2, oslot).wait()

                obuf[oslot] = o * rwbuf[xslot]
                o_copy(t0 + c, oslot).start()
                return s

            lax.fori_loop(0, nt, step, 0)

            @pl.when(nt >= 2)
            def _():
                o_copy(t0 + nt - 2, lax.rem(nt, 2)).wait()
            o_copy(t0 + nt - 1, lax.rem(nt + 1, 2)).wait()

        return carry

    lax.fori_loop(0, E, expert_body, 0)


def _ffn(seg_info, xsorted, w13, w2, roww):
    grid_spec = pltpu.PrefetchScalarGridSpec(
        num_scalar_prefetch=1,
        grid=(1,),
        in_specs=[
            pl.BlockSpec(memory_space=pl.ANY),
            pl.BlockSpec(memory_space=pl.ANY),
            pl.BlockSpec(memory_space=pl.ANY),
            pl.BlockSpec(memory_space=pl.ANY),
        ],
        out_specs=pl.BlockSpec(memory_space=pl.ANY),
        scratch_shapes=[
            pltpu.VMEM((2, 2 * F, D), _f32),
            pltpu.VMEM((2, D, F), _f32),
            pltpu.VMEM((3, TM, D), _f32),
            pltpu.VMEM((2, TM, D), _f32),
            pltpu.VMEM((3, TM, 1), _f32),
            pltpu.SemaphoreType.DMA((2,)),
            pltpu.SemaphoreType.DMA((2,)),
            pltpu.SemaphoreType.DMA((3,)),
            pltpu.SemaphoreType.DMA((2,)),
            pltpu.SemaphoreType.DMA((3,)),
        ],
    )
    return pl.pallas_call(
        _ffn_kernel,
        grid_spec=grid_spec,
        out_shape=jax.ShapeDtypeStruct((NROWS, D), _f32),
    )(seg_info, xsorted, w13, w2, roww)


def _combine_body(y_hbm, pos_hbm, out_hbm, posv, rows_v, obuf, sem):
    wid = _wid()
    pltpu.sync_copy(
        pos_hbm.at[pl.ds(pl.multiple_of(wid * K * TPW, K * TPW), K * TPW)],
        posv)
    nch = K * TPW // 32

    def chunk_gather(c):
        return pltpu.make_async_copy(
            y_hbm.at[posv.at[pl.ds(c * 32, 32)]],
            rows_v.at[c % 2], sem.at[c % 2])

    chunk_gather(0).start()
    for c in range(nch):
        if c + 1 < nch:
            chunk_gather(c + 1).start()
        chunk_gather(c).wait()
        rv = rows_v.at[c % 2]

        @plsc.parallel_loop(0, 16 * (D // L), unroll=8)
        def addloop(i):
            p = i >> 6
            dd = (i & 63) * L
            a = rv[p * 2, pl.ds(dd, L)]
            b = rv[p * 2 + 1, pl.ds(dd, L)]
            obuf[p, pl.ds(dd, L)] = a + b

        pltpu.sync_copy(
            obuf, out_hbm.at[pl.ds(pl.multiple_of(wid * TPW + c * 16, 16), 16)])


def _combine(y, pos):
    return pl.kernel(
        _combine_body,
        out_type=jax.ShapeDtypeStruct((T, D), _f32),
        mesh=_sc_mesh(),
        compiler_params=pltpu.CompilerParams(needs_layout_passes=False),
        scratch_types=[
            pltpu.VMEM((K * TPW,), _i32),
            pltpu.VMEM((2, 32, D), _f32),
            pltpu.VMEM((16, D), _f32),
            pltpu.SemaphoreType.DMA((2,)),
        ],
    )(y, pos)


@jax.jit
def kernel(hidden_states, gate_weight, w13_weight, w2_weight):
    idx, w, cnt = _router(hidden_states, gate_weight)
    cnt128 = cnt.reshape(NB * E)
    eflat = idx.reshape(NA)
    wflat = w.reshape(NA)
    rowflat, roww, seg_info = _dispatch(cnt128, eflat, wflat)
    xsorted, pos = _gather_rows(hidden_states, rowflat)
    y = _ffn(seg_info, xsorted, w13_weight, w2_weight,
             roww.reshape(NROWS, 1))
    return _combine(y, pos)
